# baseline (speedup 1.0000x reference)
import functools

import jax
import jax.numpy as jnp
from jax.experimental import pallas as pl
from jax.experimental.pallas import tpu as pltpu

B, S, D = 64, 2048, 1024
OUT = 1024
WSZ = 128
WL = 2 * WSZ + 1
H = 512
STD2 = (WSZ / 2.0) ** 2
G = 8
NB = 32


def _prep_kernel(maskf_ref, in_ref, win_ref, wfc1_ref, bfc1_ref, wfc2_ref,
                 bfc2_ref, x_ref, p_ref, len_ref, ws_ref, we_ref, cs_ref):
    maskf = maskf_ref[...]
    lengths = jnp.sum(maskf, axis=1, keepdims=True)
    x = jax.lax.dot_general(in_ref[...], win_ref[...],
                            (((1,), (1,)), ((), ())),
                            preferred_element_type=jnp.float32)
    h = jnp.tanh(jax.lax.dot_general(x, wfc1_ref[...],
                                     (((1,), (1,)), ((), ())),
                                     preferred_element_type=jnp.float32)
                 + bfc1_ref[...])
    z = jax.lax.dot_general(h, wfc2_ref[...], (((1,), (1,)), ((), ())),
                            preferred_element_type=jnp.float32)
    s = jax.nn.sigmoid(z[:, 0:1] + bfc2_ref[0])
    p = WSZ + lengths * s
    ws = jnp.round(p - WSZ).astype(jnp.int32)
    x_ref[...] = x
    p_ref[...] = p
    len_ref[...] = lengths
    ws_ref[...] = ws
    we_ref[...] = ws + WL
    cs_ref[...] = jnp.clip(ws, 0, S - WL)


def _attend_kernel(cs_ref, us_ref, p_ref, len_ref, x_ref, src_ref,
                   scores_ref, c_ref, sel, sem):
    i = pl.program_id(0)

    def window_copy(w, slot):
        return pltpu.make_async_copy(
            src_ref.at[pl.ds(cs_ref[w], WL), pl.ds(w, 1), :],
            sel.at[slot], sem.at[slot])

    def start_window(w, slot):
        window_copy(w, slot).start()

    @pl.when(i == 0)
    def _():
        for w in range(NB - G):
            start_window(w, w)

    for g in range(G):
        w2 = i * G + (NB - G) + g

        @pl.when(w2 < B)
        def _(w2=w2):
            start_window(w2, jax.lax.rem(w2, NB))

    for g in range(G):
        w = i * G + g
        window_copy(w, jax.lax.rem(w, NB)).wait()

    xall = x_ref[...]

    def compute(selms):
        base = (jax.lax.broadcasted_iota(jnp.int32, (G, WL), 1)
                .astype(jnp.float32))
        us = jnp.concatenate(
            [jnp.full((1, WL), us_ref[i * G + g].astype(jnp.float32))
             for g in range(G)], 0)
        pb = jnp.concatenate(
            [jnp.full((1, WL), p_ref[i * G + g]) for g in range(G)], 0)
        lb = jnp.concatenate(
            [jnp.full((1, WL), len_ref[i * G + g]) for g in range(G)], 0)
        pos = base + us
        gauss = jnp.exp(-((pos - pb) ** 2) / (2.0 * STD2))
        valid = (pos >= WSZ) & (pos < lb + WSZ)
        score = jnp.concatenate(
            [jax.lax.dot_general(xall[g], selms[g], (((1,), (1,)), ((), ())),
                                 preferred_element_type=jnp.float32)
             for g in range(G)], 0)
        score = jnp.where(valid, score, 1e-14)
        m = jnp.max(score, axis=1, keepdims=True)
        e = jnp.exp(score - m)
        a = (e / jnp.sum(e, axis=1, keepdims=True)) * gauss
        scores_ref[...] = a.reshape(G, 1, WL)
        for g in range(G):
            c_ref[pl.ds(g, 1)] = jax.lax.dot_general(
                a[g:g + 1], selms[g], (((1,), (0,)), ((), ())),
                preferred_element_type=jnp.float32).reshape(1, 1, D)

    compute([sel[jax.lax.rem(i * G + g, NB), :, 0, :] for g in range(G)])


def _out_kernel(c_ref, in_ref, wout_ref, out_ref):
    w = wout_ref[...]
    acc = jax.lax.dot_general(c_ref[...], w[:, :D],
                              (((1,), (1,)), ((), ())),
                              preferred_element_type=jnp.float32)
    acc += jax.lax.dot_general(in_ref[...], w[:, D:],
                               (((1,), (1,)), ((), ())),
                               preferred_element_type=jnp.float32)
    out_ref[...] = jnp.tanh(acc)


@functools.partial(jax.jit, static_argnames=("interpret",))
def _run(input, source_hids, encoder_padding_mask, W_in, W_out, W_fc1,
         b_fc1, W_fc2, b_fc2, interpret=False):
    maskf = jnp.where(encoder_padding_mask, 0.0, 1.0).astype(jnp.float32).T
    wfc2_pad = jnp.zeros((128, H), jnp.float32).at[0].set(W_fc2[0])
    x, p, lengths, ws, we, cs = pl.pallas_call(
        _prep_kernel,
        in_specs=[pl.BlockSpec(memory_space=pltpu.VMEM)] * 6
        + [pl.BlockSpec(memory_space=pltpu.SMEM)],
        out_shape=[
            jax.ShapeDtypeStruct((B, D), jnp.float32),
            jax.ShapeDtypeStruct((B, 1), jnp.float32),
            jax.ShapeDtypeStruct((B, 1), jnp.float32),
            jax.ShapeDtypeStruct((B, 1), jnp.int32),
            jax.ShapeDtypeStruct((B, 1), jnp.int32),
            jax.ShapeDtypeStruct((B, 1), jnp.int32),
        ],
        interpret=interpret,
    )(maskf, input, W_in, W_fc1, b_fc1.reshape(1, H), wfc2_pad,
      b_fc2.reshape(1))

    smem = functools.partial(pl.BlockSpec, memory_space=pltpu.SMEM)
    scores, c = pl.pallas_call(
        _attend_kernel,
        grid=(B // G,),
        in_specs=[
            smem(),
            smem(),
            smem(),
            smem(),
            pl.BlockSpec((G, 1, D), lambda i: (i, 0, 0)),
            pl.BlockSpec(memory_space=pl.ANY),
        ],
        out_specs=[
            pl.BlockSpec((G, 1, WL), lambda i: (i, 0, 0)),
            pl.BlockSpec((G, 1, D), lambda i: (i, 0, 0)),
        ],
        out_shape=[
            jax.ShapeDtypeStruct((B, 1, WL), jnp.float32),
            jax.ShapeDtypeStruct((B, 1, D), jnp.float32),
        ],
        scratch_shapes=[
            pltpu.VMEM((NB, WL, 1, D), jnp.float32),
            pltpu.SemaphoreType.DMA((NB,)),
        ],
        interpret=interpret,
    )(cs.reshape(B), ws.reshape(B), p.reshape(B), lengths.reshape(B),
      x.reshape(B, 1, D), source_hids)
    scores = scores.reshape(B, WL)
    c = c.reshape(B, D)

    out = pl.pallas_call(
        _out_kernel,
        out_shape=jax.ShapeDtypeStruct((B, OUT), jnp.float32),
        interpret=interpret,
    )(c, input, W_out)
    return out, scores, ws, we


def kernel(input, source_hids, encoder_padding_mask, W_in, W_out, W_fc1,
           b_fc1, W_fc2, b_fc2):
    return _run(input, source_hids, encoder_padding_mask, W_in, W_out,
                W_fc1, b_fc1, W_fc2, b_fc2)


# speedup vs baseline: 1.0452x; 1.0452x over previous
import functools

import jax
import jax.numpy as jnp
from jax.experimental import pallas as pl
from jax.experimental.pallas import tpu as pltpu

B, S, D = 64, 2048, 1024
OUT = 1024
WSZ = 128
WL = 2 * WSZ + 1
H = 512
STD2 = (WSZ / 2.0) ** 2
G = 8
NB = 24


def _prep_kernel(maskf_ref, in_ref, win_ref, wfc1_ref, bfc1_ref, wfc2_ref,
                 bfc2_ref, x_ref, p_ref, len_ref, ws_ref, we_ref, cs_ref):
    maskf = maskf_ref[...]
    lengths = jnp.sum(maskf, axis=1, keepdims=True)
    x = jax.lax.dot_general(in_ref[...], win_ref[...],
                            (((1,), (1,)), ((), ())),
                            preferred_element_type=jnp.float32)
    h = jnp.tanh(jax.lax.dot_general(x, wfc1_ref[...],
                                     (((1,), (1,)), ((), ())),
                                     preferred_element_type=jnp.float32)
                 + bfc1_ref[...])
    z = jax.lax.dot_general(h, wfc2_ref[...], (((1,), (1,)), ((), ())),
                            preferred_element_type=jnp.float32)
    s = jax.nn.sigmoid(z[:, 0:1] + bfc2_ref[0])
    p = WSZ + lengths * s
    ws = jnp.round(p - WSZ).astype(jnp.int32)
    x_ref[...] = x
    p_ref[...] = p
    len_ref[...] = lengths
    ws_ref[...] = ws
    we_ref[...] = ws + WL
    cs_ref[...] = jnp.clip(ws, 0, S - WL)


def _attend_kernel(cs_ref, us_ref, p_ref, len_ref, x_ref, src_ref,
                   scores_ref, c_ref, sel, sem):
    i = pl.program_id(0)

    def window_copy(w, slot):
        return pltpu.make_async_copy(
            src_ref.at[pl.ds(cs_ref[w], WL), pl.ds(w, 1), :],
            sel.at[slot], sem.at[slot])

    def start_window(w, slot):
        window_copy(w, slot).start()

    @pl.when(i == 0)
    def _():
        for w in range(NB - G):
            start_window(w, w)

    for g in range(G):
        w2 = i * G + (NB - G) + g

        @pl.when(w2 < B)
        def _(w2=w2):
            start_window(w2, jax.lax.rem(w2, NB))

    for g in range(G):
        w = i * G + g
        window_copy(w, jax.lax.rem(w, NB)).wait()

    xall = x_ref[...]

    def compute(selms):
        base = (jax.lax.broadcasted_iota(jnp.int32, (G, WL), 1)
                .astype(jnp.float32))
        us = jnp.concatenate(
            [jnp.full((1, WL), us_ref[i * G + g].astype(jnp.float32))
             for g in range(G)], 0)
        pb = jnp.concatenate(
            [jnp.full((1, WL), p_ref[i * G + g]) for g in range(G)], 0)
        lb = jnp.concatenate(
            [jnp.full((1, WL), len_ref[i * G + g]) for g in range(G)], 0)
        pos = base + us
        gauss = jnp.exp(-((pos - pb) ** 2) / (2.0 * STD2))
        valid = (pos >= WSZ) & (pos < lb + WSZ)
        score = jnp.concatenate(
            [jax.lax.dot_general(xall[g], selms[g], (((1,), (1,)), ((), ())),
                                 preferred_element_type=jnp.float32)
             for g in range(G)], 0)
        score = jnp.where(valid, score, 1e-14)
        m = jnp.max(score, axis=1, keepdims=True)
        e = jnp.exp(score - m)
        a = (e / jnp.sum(e, axis=1, keepdims=True)) * gauss
        scores_ref[...] = a.reshape(G, 1, WL)
        for g in range(G):
            c_ref[pl.ds(g, 1)] = jax.lax.dot_general(
                a[g:g + 1], selms[g], (((1,), (0,)), ((), ())),
                preferred_element_type=jnp.float32).reshape(1, 1, D)

    compute([sel[jax.lax.rem(i * G + g, NB), :, 0, :] for g in range(G)])


def _out_kernel(c_ref, in_ref, wout_ref, out_ref):
    w = wout_ref[...]
    acc = jax.lax.dot_general(c_ref[...], w[:, :D],
                              (((1,), (1,)), ((), ())),
                              preferred_element_type=jnp.float32)
    acc += jax.lax.dot_general(in_ref[...], w[:, D:],
                               (((1,), (1,)), ((), ())),
                               preferred_element_type=jnp.float32)
    out_ref[...] = jnp.tanh(acc)


@functools.partial(jax.jit, static_argnames=("interpret",))
def _run(input, source_hids, encoder_padding_mask, W_in, W_out, W_fc1,
         b_fc1, W_fc2, b_fc2, interpret=False):
    maskf = jnp.where(encoder_padding_mask, 0.0, 1.0).astype(jnp.float32).T
    wfc2_pad = jnp.zeros((128, H), jnp.float32).at[0].set(W_fc2[0])
    x, p, lengths, ws, we, cs = pl.pallas_call(
        _prep_kernel,
        in_specs=[pl.BlockSpec(memory_space=pltpu.VMEM)] * 6
        + [pl.BlockSpec(memory_space=pltpu.SMEM)],
        out_shape=[
            jax.ShapeDtypeStruct((B, D), jnp.float32),
            jax.ShapeDtypeStruct((B, 1), jnp.float32),
            jax.ShapeDtypeStruct((B, 1), jnp.float32),
            jax.ShapeDtypeStruct((B, 1), jnp.int32),
            jax.ShapeDtypeStruct((B, 1), jnp.int32),
            jax.ShapeDtypeStruct((B, 1), jnp.int32),
        ],
        interpret=interpret,
    )(maskf, input, W_in, W_fc1, b_fc1.reshape(1, H), wfc2_pad,
      b_fc2.reshape(1))

    smem = functools.partial(pl.BlockSpec, memory_space=pltpu.SMEM)
    scores, c = pl.pallas_call(
        _attend_kernel,
        grid=(B // G,),
        in_specs=[
            smem(),
            smem(),
            smem(),
            smem(),
            pl.BlockSpec((G, 1, D), lambda i: (i, 0, 0)),
            pl.BlockSpec(memory_space=pl.ANY),
        ],
        out_specs=[
            pl.BlockSpec((G, 1, WL), lambda i: (i, 0, 0)),
            pl.BlockSpec((G, 1, D), lambda i: (i, 0, 0)),
        ],
        out_shape=[
            jax.ShapeDtypeStruct((B, 1, WL), jnp.float32),
            jax.ShapeDtypeStruct((B, 1, D), jnp.float32),
        ],
        scratch_shapes=[
            pltpu.VMEM((NB, WL, 1, D), jnp.float32),
            pltpu.SemaphoreType.DMA((NB,)),
        ],
        interpret=interpret,
    )(cs.reshape(B), ws.reshape(B), p.reshape(B), lengths.reshape(B),
      x.reshape(B, 1, D), source_hids)
    scores = scores.reshape(B, WL)
    c = c.reshape(B, D)

    out = pl.pallas_call(
        _out_kernel,
        out_shape=jax.ShapeDtypeStruct((B, OUT), jnp.float32),
        interpret=interpret,
    )(c, input, W_out)
    return out, scores, ws, we


def kernel(input, source_hids, encoder_padding_mask, W_in, W_out, W_fc1,
           b_fc1, W_fc2, b_fc2):
    return _run(input, source_hids, encoder_padding_mask, W_in, W_out,
                W_fc1, b_fc1, W_fc2, b_fc2)
